# baseline (device time: 312546 ns/iter reference)
import jax
import jax.numpy as jnp
from jax import lax
from jax.experimental import pallas as pl
from jax.experimental.pallas import tpu as pltpu

N_DEV = 4


def kernel(x, w_mat):
    m, k = x.shape
    _, n = w_mat.shape
    ch = m // N_DEV

    def body(x_ref, w_ref, out_ref, rs_recv, send_sems, recv_sems):
        my = lax.axis_index("i")
        left = lax.rem(my + N_DEV - 1, N_DEV)
        right = lax.rem(my + 1, N_DEV)

        barrier_sem = pltpu.get_barrier_semaphore()
        for nbr in (left, right):
            pl.semaphore_signal(
                barrier_sem, inc=1,
                device_id=(nbr,), device_id_type=pl.DeviceIdType.MESH,
            )
        pl.semaphore_wait(barrier_sem, 2)

        out_ref[:, :] = jnp.dot(
            x_ref[:, :], w_ref[:, :], preferred_element_type=jnp.float32
        )

        for s in range(N_DEV - 1):
            send_c = lax.rem(my - s + 2 * N_DEV, N_DEV)
            recv_c = lax.rem(my - s - 1 + 2 * N_DEV, N_DEV)
            rdma = pltpu.make_async_remote_copy(
                src_ref=out_ref.at[pl.ds(send_c * ch, ch), :],
                dst_ref=rs_recv.at[s],
                send_sem=send_sems.at[s],
                recv_sem=recv_sems.at[s],
                device_id=(right,),
                device_id_type=pl.DeviceIdType.MESH,
            )
            rdma.start()
            rdma.wait()
            out_ref[pl.ds(recv_c * ch, ch), :] = (
                out_ref[pl.ds(recv_c * ch, ch), :] + rs_recv[s]
            )

        for a in range(N_DEV - 1):
            h = (N_DEV - 1) + a
            send_c = lax.rem(my + 1 - a + 2 * N_DEV, N_DEV)
            rdma = pltpu.make_async_remote_copy(
                src_ref=out_ref.at[pl.ds(send_c * ch, ch), :],
                dst_ref=out_ref.at[pl.ds(send_c * ch, ch), :],
                send_sem=send_sems.at[h],
                recv_sem=recv_sems.at[h],
                device_id=(right,),
                device_id_type=pl.DeviceIdType.MESH,
            )
            rdma.start()
            rdma.wait()

        out_ref[:, :] = jnp.maximum(out_ref[:, :], 0.0)

    return pl.pallas_call(
        body,
        out_shape=jax.ShapeDtypeStruct((m, n), jnp.float32),
        in_specs=[
            pl.BlockSpec(memory_space=pltpu.VMEM),
            pl.BlockSpec(memory_space=pltpu.VMEM),
        ],
        out_specs=pl.BlockSpec(memory_space=pltpu.VMEM),
        scratch_shapes=[
            pltpu.VMEM((N_DEV - 1, ch, n), jnp.float32),
            pltpu.SemaphoreType.DMA((2 * (N_DEV - 1),)),
            pltpu.SemaphoreType.DMA((2 * (N_DEV - 1),)),
        ],
        compiler_params=pltpu.CompilerParams(collective_id=0),
    )(x, w_mat)


# device time: 106877 ns/iter; 2.9244x vs baseline; 2.9244x over previous
import jax
import jax.numpy as jnp
from jax import lax
from jax.experimental import pallas as pl
from jax.experimental.pallas import tpu as pltpu

N_DEV = 4


def kernel(x, w_mat):
    m, k = x.shape
    _, n = w_mat.shape
    hm = m // 2
    qm = m // 4
    hn = n // 2

    cA = pl.ds(0, hn)
    cB = pl.ds(hn, hn)

    def body(x_ref, w_ref, out_ref,
             sA1, sB1, rA1, rB1, sA2, sB2, rA2, rB2,
             agA, agB,
             semA_s, semA_r, semB_s, semB_r):
        rA4, rB4 = sA1, sB1
        my = lax.axis_index("i")
        pY = my ^ 1
        pX = 3 - my

        barrier_sem = pltpu.get_barrier_semaphore()
        for nbr in (pY, pX):
            pl.semaphore_signal(
                barrier_sem, inc=1,
                device_id=(nbr,), device_id_type=pl.DeviceIdType.MESH,
            )
        pl.semaphore_wait(barrier_sem, 2)

        qA = my
        hA = my // 2
        relA = my % 2
        qB = jnp.where(my == 0, 0, jnp.where(my == 1, 2, jnp.where(my == 2, 3, 1)))
        qBx = jnp.where(my == 0, 1, jnp.where(my == 1, 3, jnp.where(my == 2, 2, 0)))
        hB = jnp.where((my == 1) | (my == 2), 1, 0)
        relB = qB - 2 * hB

        out_ref[:, :] = jnp.dot(
            x_ref[:, :].astype(jnp.bfloat16),
            w_ref[:, :].astype(jnp.bfloat16),
            preferred_element_type=jnp.float32,
        )

        def rdma(src, dst, sems_s, sems_r, h, dev):
            return pltpu.make_async_remote_copy(
                src_ref=src, dst_ref=dst,
                send_sem=sems_s.at[h], recv_sem=sems_r.at[h],
                device_id=(dev,), device_id_type=pl.DeviceIdType.MESH,
            )

        sA1[:, :] = out_ref[pl.ds((1 - hA) * hm, hm), cA].astype(jnp.bfloat16)
        sB1[:, :] = out_ref[pl.ds((1 - hB) * hm, hm), cB].astype(jnp.bfloat16)
        a1 = rdma(sA1, rA1, semA_s, semA_r, 0, pX)
        b1 = rdma(sB1, rB1, semB_s, semB_r, 0, pY)
        a1.start()
        b1.start()
        a1.wait()
        b1.wait()
        out_ref[pl.ds(hA * hm, hm), cA] = (
            out_ref[pl.ds(hA * hm, hm), cA] + rA1[:, :].astype(jnp.float32)
        )
        out_ref[pl.ds(hB * hm, hm), cB] = (
            out_ref[pl.ds(hB * hm, hm), cB] + rB1[:, :].astype(jnp.float32)
        )

        sA2[:, :] = out_ref[pl.ds(pY * qm, qm), cA].astype(jnp.bfloat16)
        sB2[:, :] = out_ref[pl.ds(qBx * qm, qm), cB].astype(jnp.bfloat16)
        a2 = rdma(sA2, rA2, semA_s, semA_r, 1, pY)
        b2 = rdma(sB2, rB2, semB_s, semB_r, 1, pX)
        a2.start()
        b2.start()
        a2.wait()
        b2.wait()

        qa = jnp.maximum(
            out_ref[pl.ds(qA * qm, qm), cA] + rA2[:, :].astype(jnp.float32), 0.0
        )
        out_ref[pl.ds(qA * qm, qm), cA] = qa
        agA[pl.ds(relA * qm, qm), :] = qa.astype(jnp.bfloat16)
        qb = jnp.maximum(
            out_ref[pl.ds(qB * qm, qm), cB] + rB2[:, :].astype(jnp.float32), 0.0
        )
        out_ref[pl.ds(qB * qm, qm), cB] = qb
        agB[pl.ds(relB * qm, qm), :] = qb.astype(jnp.bfloat16)

        a3 = rdma(agA.at[pl.ds(relA * qm, qm), :],
                  agA.at[pl.ds(relA * qm, qm), :], semA_s, semA_r, 2, pY)
        b3 = rdma(agB.at[pl.ds(relB * qm, qm), :],
                  agB.at[pl.ds(relB * qm, qm), :], semB_s, semB_r, 2, pX)
        a3.start()
        b3.start()
        a3.wait()
        b3.wait()
        out_ref[pl.ds(pY * qm, qm), cA] = (
            agA[pl.ds((1 - relA) * qm, qm), :].astype(jnp.float32)
        )
        out_ref[pl.ds(qBx * qm, qm), cB] = (
            agB[pl.ds((1 - relB) * qm, qm), :].astype(jnp.float32)
        )

        a4 = rdma(agA, rA4, semA_s, semA_r, 3, pX)
        b4 = rdma(agB, rB4, semB_s, semB_r, 3, pY)
        a4.start()
        b4.start()
        a4.wait()
        b4.wait()
        out_ref[pl.ds((1 - hA) * hm, hm), cA] = rA4[:, :].astype(jnp.float32)
        out_ref[pl.ds((1 - hB) * hm, hm), cB] = rB4[:, :].astype(jnp.float32)

    bf = jnp.bfloat16
    return pl.pallas_call(
        body,
        out_shape=jax.ShapeDtypeStruct((m, n), jnp.float32),
        in_specs=[
            pl.BlockSpec(memory_space=pltpu.VMEM),
            pl.BlockSpec(memory_space=pltpu.VMEM),
        ],
        out_specs=pl.BlockSpec(memory_space=pltpu.VMEM),
        scratch_shapes=[
            pltpu.VMEM((hm, hn), bf),
            pltpu.VMEM((hm, hn), bf),
            pltpu.VMEM((hm, hn), bf),
            pltpu.VMEM((hm, hn), bf),
            pltpu.VMEM((qm, hn), bf),
            pltpu.VMEM((qm, hn), bf),
            pltpu.VMEM((qm, hn), bf),
            pltpu.VMEM((qm, hn), bf),
            pltpu.VMEM((hm, hn), bf),
            pltpu.VMEM((hm, hn), bf),
            pltpu.SemaphoreType.DMA((4,)),
            pltpu.SemaphoreType.DMA((4,)),
            pltpu.SemaphoreType.DMA((4,)),
            pltpu.SemaphoreType.DMA((4,)),
        ],
        compiler_params=pltpu.CompilerParams(
            collective_id=0, vmem_limit_bytes=64 * 1024 * 1024
        ),
    )(x, w_mat)


# device time: 96413 ns/iter; 3.2417x vs baseline; 1.1085x over previous
import jax
import jax.numpy as jnp
from jax import lax
from jax.experimental import pallas as pl
from jax.experimental.pallas import tpu as pltpu

N_DEV = 4


def kernel(x, w_mat):
    m, k = x.shape
    _, n = w_mat.shape
    hm = m // 2
    qm = m // 4
    hn = n // 2
    sn = n // 4

    def body(x_ref, w_ref, out_ref,
             xbf, wbf,
             s1A, s1B, r1A, r1B, s2A, s2B, r2A, r2B, agA, agB,
             sem_s, sem_r):
        r4A, r4B = s1A, s1B

        my = lax.axis_index("i")
        pY = my ^ 1
        pX = 3 - my

        barrier_sem = pltpu.get_barrier_semaphore()
        for nbr in (pY, pX):
            pl.semaphore_signal(
                barrier_sem, inc=1,
                device_id=(nbr,), device_id_type=pl.DeviceIdType.MESH,
            )
        pl.semaphore_wait(barrier_sem, 2)

        hA = my // 2
        relA = my % 2
        qB = jnp.where(my == 0, 0, jnp.where(my == 1, 2, jnp.where(my == 2, 3, 1)))
        qBx = jnp.where(my == 0, 1, jnp.where(my == 1, 3, jnp.where(my == 2, 2, 0)))
        hB = jnp.where((my == 1) | (my == 2), 1, 0)
        relB = qB - 2 * hB

        xbf[:, :] = x_ref[:, :].astype(jnp.bfloat16)
        wbf[:, :] = w_ref[:, :].astype(jnp.bfloat16)

        strips = [
            (0, pl.ds(0, sn), pl.ds(0, sn),
             s1A, r1A, s2A, r2A, agA, r4A, hA, relA, my, pY, pX, pY),
            (4, pl.ds(sn, sn), pl.ds(sn, sn),
             s1A, r1A, s2A, r2A, agA, r4A, hA, relA, my, pY, pX, pY),
            (8, pl.ds(hn, sn), pl.ds(0, sn),
             s1B, r1B, s2B, r2B, agB, r4B, hB, relB, qB, qBx, pY, pX),
            (12, pl.ds(hn + sn, sn), pl.ds(sn, sn),
             s1B, r1B, s2B, r2B, agB, r4B, hB, relB, qB, qBx, pY, pX),
        ]
        order = (0, 2, 1, 3)

        def rdma(src, dst, sem, dev):
            return pltpu.make_async_remote_copy(
                src_ref=src, dst_ref=dst,
                send_sem=sem_s.at[sem], recv_sem=sem_r.at[sem],
                device_id=(dev,), device_id_type=pl.DeviceIdType.MESH,
            )

        ds_ = pl.ds
        rs1 = {}
        for i in order:
            sem, c, cs, s1, r1, _, _, _, _, h, _, _, _, d1, _ = strips[i]
            s1[:, cs] = jnp.dot(
                xbf[ds_((1 - h) * hm, hm), :], wbf[:, c],
                preferred_element_type=jnp.float32,
            ).astype(jnp.bfloat16)
            rs1[i] = rdma(s1.at[:, cs], r1.at[:, cs], sem + 0, d1)
            rs1[i].start()

        out_ref[ds_(hA * hm, hm), ds_(0, hn)] = jnp.dot(
            xbf[ds_(hA * hm, hm), :], wbf[:, ds_(0, hn)],
            preferred_element_type=jnp.float32,
        )
        out_ref[ds_(hB * hm, hm), ds_(hn, hn)] = jnp.dot(
            xbf[ds_(hB * hm, hm), :], wbf[:, ds_(hn, hn)],
            preferred_element_type=jnp.float32,
        )

        rs2 = {}
        for i in order:
            sem, c, cs, _, r1, s2, r2, _, _, h, rel, _, pq, _, d2 = strips[i]
            rs1[i].wait_recv()
            s2[:, cs] = (
                out_ref[ds_(pq * qm, qm), c]
                + r1[ds_((1 - rel) * qm, qm), cs].astype(jnp.float32)
            ).astype(jnp.bfloat16)
            rs2[i] = rdma(s2.at[:, cs], r2.at[:, cs], sem + 1, d2)
            rs2[i].start()
        for i in order:
            _, c, cs, _, r1, _, _, _, _, _, rel, q, _, _, _ = strips[i]
            out_ref[ds_(q * qm, qm), c] = (
                out_ref[ds_(q * qm, qm), c]
                + r1[ds_(rel * qm, qm), cs].astype(jnp.float32)
            )

        ag1 = {}
        for i in order:
            sem, c, cs, _, _, _, r2, ag, _, _, rel, q, _, _, d2 = strips[i]
            rs2[i].wait_recv()
            qv = jnp.maximum(
                out_ref[ds_(q * qm, qm), c] + r2[:, cs].astype(jnp.float32),
                0.0,
            )
            out_ref[ds_(q * qm, qm), c] = qv
            ag[ds_(rel * qm, qm), cs] = qv.astype(jnp.bfloat16)
            ag1[i] = rdma(
                ag.at[ds_(rel * qm, qm), cs], ag.at[ds_(rel * qm, qm), cs],
                sem + 2, d2,
            )
            ag1[i].start()

        ag2 = {}
        for i in order:
            sem, c, cs, _, _, _, _, ag, r4, _, rel, _, pq, d1, _ = strips[i]
            ag1[i].wait_recv()
            ag2[i] = rdma(ag.at[:, cs], r4.at[:, cs], sem + 3, d1)
            ag2[i].start()
            out_ref[ds_(pq * qm, qm), c] = (
                ag[ds_((1 - rel) * qm, qm), cs].astype(jnp.float32)
            )

        for i in order:
            _, c, cs, _, _, _, _, _, r4, h, _, _, _, _, _ = strips[i]
            ag2[i].wait_recv()
            out_ref[ds_((1 - h) * hm, hm), c] = r4[:, cs].astype(jnp.float32)
        for i in order:
            for d in (rs1, rs2, ag1, ag2):
                d[i].wait_send()

    bf = jnp.bfloat16
    return pl.pallas_call(
        body,
        out_shape=jax.ShapeDtypeStruct((m, n), jnp.float32),
        in_specs=[
            pl.BlockSpec(memory_space=pltpu.VMEM),
            pl.BlockSpec(memory_space=pltpu.VMEM),
        ],
        out_specs=pl.BlockSpec(memory_space=pltpu.VMEM),
        scratch_shapes=[
            pltpu.VMEM((m, k), bf),
            pltpu.VMEM((k, n), bf),
            pltpu.VMEM((hm, hn), bf),
            pltpu.VMEM((hm, hn), bf),
            pltpu.VMEM((hm, hn), bf),
            pltpu.VMEM((hm, hn), bf),
            pltpu.VMEM((qm, hn), bf),
            pltpu.VMEM((qm, hn), bf),
            pltpu.VMEM((qm, hn), bf),
            pltpu.VMEM((qm, hn), bf),
            pltpu.VMEM((hm, hn), bf),
            pltpu.VMEM((hm, hn), bf),
            pltpu.SemaphoreType.DMA((16,)),
            pltpu.SemaphoreType.DMA((16,)),
        ],
        compiler_params=pltpu.CompilerParams(
            collective_id=0, vmem_limit_bytes=64 * 1024 * 1024
        ),
    )(x, w_mat)


# device time: 95419 ns/iter; 3.2755x vs baseline; 1.0104x over previous
import jax
import jax.numpy as jnp
from jax import lax
from jax.experimental import pallas as pl
from jax.experimental.pallas import tpu as pltpu

N_DEV = 4


def kernel(x, w_mat):
    m, k = x.shape
    _, n = w_mat.shape
    hm = m // 2
    qm = m // 4
    hn = n // 2
    sn = n // 4

    def body(x_ref, w_ref, out_ref,
             xbf, wbf,
             s1A, s1B, r1A, r1B, s2A, s2B, r2A, r2B, agA, agB,
             sem_s, sem_r, readyY):
        r4A, r4B = s1A, s1B

        my = lax.axis_index("i")
        pY = my ^ 1
        pX = 3 - my

        barrier_sem = pltpu.get_barrier_semaphore()
        pl.semaphore_signal(
            barrier_sem, inc=1,
            device_id=(pX,), device_id_type=pl.DeviceIdType.MESH,
        )
        pl.semaphore_signal(
            readyY, inc=1,
            device_id=(pY,), device_id_type=pl.DeviceIdType.MESH,
        )

        hA = my // 2
        relA = my % 2
        qB = jnp.where(my == 0, 0, jnp.where(my == 1, 2, jnp.where(my == 2, 3, 1)))
        qBx = jnp.where(my == 0, 1, jnp.where(my == 1, 3, jnp.where(my == 2, 2, 0)))
        hB = jnp.where((my == 1) | (my == 2), 1, 0)
        relB = qB - 2 * hB

        xbf[:, :] = x_ref[:, :].astype(jnp.bfloat16)
        wbf[:, :] = w_ref[:, :].astype(jnp.bfloat16)

        strips = [
            (0, pl.ds(0, sn), pl.ds(0, sn),
             s1A, r1A, s2A, r2A, agA, r4A, hA, relA, my, pY, pX, pY),
            (4, pl.ds(sn, sn), pl.ds(sn, sn),
             s1A, r1A, s2A, r2A, agA, r4A, hA, relA, my, pY, pX, pY),
            (8, pl.ds(hn, sn), pl.ds(0, sn),
             s1B, r1B, s2B, r2B, agB, r4B, hB, relB, qB, qBx, pY, pX),
            (12, pl.ds(hn + sn, sn), pl.ds(sn, sn),
             s1B, r1B, s2B, r2B, agB, r4B, hB, relB, qB, qBx, pY, pX),
        ]
        order = (0, 2, 1, 3)

        def rdma(src, dst, sem, dev):
            return pltpu.make_async_remote_copy(
                src_ref=src, dst_ref=dst,
                send_sem=sem_s.at[sem], recv_sem=sem_r.at[sem],
                device_id=(dev,), device_id_type=pl.DeviceIdType.MESH,
            )

        ds_ = pl.ds
        rs1 = {}
        waited = {pX_key: False for pX_key in ("x", "y")}
        for i in order:
            sem, c, cs, s1, r1, _, _, _, _, h, _, _, _, d1, _ = strips[i]
            s1[:, cs] = jnp.dot(
                xbf[ds_((1 - h) * hm, hm), :], wbf[:, c],
                preferred_element_type=jnp.float32,
            ).astype(jnp.bfloat16)
            link = "x" if i < 2 else "y"
            if not waited[link]:
                if link == "x":
                    pl.semaphore_wait(barrier_sem, 1)
                else:
                    pl.semaphore_wait(readyY, 1)
                waited[link] = True
            rs1[i] = rdma(s1.at[:, cs], r1.at[:, cs], sem + 0, d1)
            rs1[i].start()

        out_ref[ds_(hA * hm, hm), ds_(0, hn)] = jnp.dot(
            xbf[ds_(hA * hm, hm), :], wbf[:, ds_(0, hn)],
            preferred_element_type=jnp.float32,
        )
        out_ref[ds_(hB * hm, hm), ds_(hn, hn)] = jnp.dot(
            xbf[ds_(hB * hm, hm), :], wbf[:, ds_(hn, hn)],
            preferred_element_type=jnp.float32,
        )

        rs2 = {}
        for i in order:
            sem, c, cs, _, r1, s2, r2, _, _, h, rel, _, pq, _, d2 = strips[i]
            rs1[i].wait_recv()
            s2[:, cs] = (
                out_ref[ds_(pq * qm, qm), c]
                + r1[ds_((1 - rel) * qm, qm), cs].astype(jnp.float32)
            ).astype(jnp.bfloat16)
            rs2[i] = rdma(s2.at[:, cs], r2.at[:, cs], sem + 1, d2)
            rs2[i].start()
        for i in order:
            _, c, cs, _, r1, _, _, _, _, _, rel, q, _, _, _ = strips[i]
            out_ref[ds_(q * qm, qm), c] = (
                out_ref[ds_(q * qm, qm), c]
                + r1[ds_(rel * qm, qm), cs].astype(jnp.float32)
            )

        ag1 = {}
        for i in order:
            sem, c, cs, _, _, _, r2, ag, _, _, rel, q, _, _, d2 = strips[i]
            rs2[i].wait_recv()
            qv = jnp.maximum(
                out_ref[ds_(q * qm, qm), c] + r2[:, cs].astype(jnp.float32),
                0.0,
            )
            out_ref[ds_(q * qm, qm), c] = qv
            ag[ds_(rel * qm, qm), cs] = qv.astype(jnp.bfloat16)
            ag1[i] = rdma(
                ag.at[ds_(rel * qm, qm), cs], ag.at[ds_(rel * qm, qm), cs],
                sem + 2, d2,
            )
            ag1[i].start()

        ag2 = {}
        for i in order:
            sem, c, cs, _, _, _, _, ag, r4, _, rel, _, pq, d1, _ = strips[i]
            ag1[i].wait_recv()
            ag2[i] = rdma(ag.at[:, cs], r4.at[:, cs], sem + 3, d1)
            ag2[i].start()
            out_ref[ds_(pq * qm, qm), c] = (
                ag[ds_((1 - rel) * qm, qm), cs].astype(jnp.float32)
            )

        for i in order:
            _, c, cs, _, _, _, _, _, r4, h, _, _, _, _, _ = strips[i]
            ag2[i].wait_recv()
            out_ref[ds_((1 - h) * hm, hm), c] = r4[:, cs].astype(jnp.float32)
        for i in order:
            for d in (rs1, rs2, ag1, ag2):
                d[i].wait_send()

    bf = jnp.bfloat16
    return pl.pallas_call(
        body,
        out_shape=jax.ShapeDtypeStruct((m, n), jnp.float32),
        in_specs=[
            pl.BlockSpec(memory_space=pltpu.VMEM),
            pl.BlockSpec(memory_space=pltpu.VMEM),
        ],
        out_specs=pl.BlockSpec(memory_space=pltpu.VMEM),
        scratch_shapes=[
            pltpu.VMEM((m, k), bf),
            pltpu.VMEM((k, n), bf),
            pltpu.VMEM((hm, hn), bf),
            pltpu.VMEM((hm, hn), bf),
            pltpu.VMEM((hm, hn), bf),
            pltpu.VMEM((hm, hn), bf),
            pltpu.VMEM((qm, hn), bf),
            pltpu.VMEM((qm, hn), bf),
            pltpu.VMEM((qm, hn), bf),
            pltpu.VMEM((qm, hn), bf),
            pltpu.VMEM((hm, hn), bf),
            pltpu.VMEM((hm, hn), bf),
            pltpu.SemaphoreType.DMA((16,)),
            pltpu.SemaphoreType.DMA((16,)),
            pltpu.SemaphoreType.REGULAR,
        ],
        compiler_params=pltpu.CompilerParams(
            collective_id=0, vmem_limit_bytes=64 * 1024 * 1024
        ),
    )(x, w_mat)
